# baseline (speedup 1.0000x reference)
import functools

from jax._src import config as _jax_config

import jax
import jax.numpy as jnp
from jax import lax
from jax.experimental import pallas as pl
from jax.experimental.pallas import tpu as pltpu
from jax.experimental.pallas import tpu_sc as plsc


def _make_sc_fusion(n_out: int, interleaved: bool, ch: int, unroll: int = 5):
    info = plsc.get_sparse_core_info()
    nc, ns, L = info.num_cores, info.num_subcores, info.num_lanes
    nw = nc * ns
    assert n_out % (nw * ch) == 0
    out_w = n_out // nw
    nch = out_w // ch
    assert nch % 2 == 0 and ch % 8 == 0 and (ch // L) % unroll == 0
    mult = 2 if interleaved else 1
    in_ch = ch * mult

    mesh = plsc.VectorSubcoreMesh(core_axis_name="c", subcore_axis_name="s")
    scratch = [
        pltpu.VMEM((in_ch,), jnp.int32),
        pltpu.VMEM((in_ch,), jnp.int32),
        pltpu.VMEM((in_ch,), jnp.int32),
        pltpu.VMEM((in_ch,), jnp.int32),
        pltpu.VMEM((ch,), jnp.float32),
        pltpu.VMEM((ch,), jnp.float32),
        pltpu.VMEM((L,), jnp.float32),
    ] + [pltpu.SemaphoreType.DMA] * 6

    @functools.partial(
        pl.kernel,
        mesh=mesh,
        out_type=jax.ShapeDtypeStruct((n_out,), jnp.float32),
        scratch_types=scratch,
        compiler_params=pltpu.CompilerParams(needs_layout_passes=False),
    )
    def fused(a0_hbm, an_hbm, wv_hbm, out_hbm,
              a0b0, a0b1, anb0, anb1, ob0, ob1, wvb,
              sa0, sa1, sn0, sn1, so0, so1):
        a0bs, anbs, obs = (a0b0, a0b1), (anb0, anb1), (ob0, ob1)
        sas, sns, sos = (sa0, sa1), (sn0, sn1), (so0, so1)
        wid = lax.axis_index("s") * jnp.int32(nc) + lax.axis_index("c")
        base = wid * jnp.int32(out_w)

        pltpu.sync_copy(wv_hbm, wvb)
        wv = wvb[...]
        omv = 1.0 - wv

        def start_in(g, b):
            off = (base + jnp.int32(g * ch)) * jnp.int32(mult)
            pltpu.async_copy(a0_hbm.at[pl.ds(off, in_ch)], a0bs[b], sas[b])
            pltpu.async_copy(an_hbm.at[pl.ds(off, in_ch)], anbs[b], sns[b])

        def wait_in(b):
            pltpu.make_async_copy(
                a0_hbm.at[pl.ds(0, in_ch)], a0bs[b], sas[b]).wait()
            pltpu.make_async_copy(
                an_hbm.at[pl.ds(0, in_ch)], anbs[b], sns[b]).wait()

        def wait_out(b):
            pltpu.make_async_copy(
                obs[b], out_hbm.at[pl.ds(0, ch)], sos[b]).wait()

        start_in(0, 0)
        start_in(1, 1)

        gather_iota = lax.broadcasted_iota(jnp.int32, (L,), 0) * jnp.int32(mult)

        def compute(b):
            a0b, anb, ob = a0bs[b], anbs[b], obs[b]

            @plsc.parallel_loop(0, ch // L, unroll=unroll)
            def _(j):
                if interleaved:
                    idx = j * jnp.int32(2 * L) + gather_iota
                    a0 = plsc.load_gather(a0b, [idx])
                    an = plsc.load_gather(anb, [idx])
                else:
                    a0 = a0b[pl.ds(j * jnp.int32(L), L)]
                    an = anb[pl.ds(j * jnp.int32(L), L)]
                ob[pl.ds(j * jnp.int32(L), L)] = (
                    a0.astype(jnp.float32) * wv + an.astype(jnp.float32) * omv)

        for g in range(nch):
            b = g % 2
            wait_in(b)
            if g >= 2:
                wait_out(b)
            compute(b)
            pltpu.async_copy(
                obs[b], out_hbm.at[pl.ds(base + g * jnp.int32(ch), ch)], sos[b])
            if g + 2 < nch:
                start_in(g + 2, b)

        for b in range(2):
            wait_out(b)

    return fused


def kernel(A0_edge_index, An_edge_index, ver, p):
    with _jax_config.enable_x64(False):
        return _kernel_impl(A0_edge_index, An_edge_index, ver, p)


def _kernel_impl(A0_edge_index, An_edge_index, ver, p):
    del ver
    shape = A0_edge_index.shape
    n_out = shape[0] * shape[1]
    w = lax.stop_gradient(jax.nn.sigmoid(p)).astype(jnp.float32)
    wv = jnp.full((16,), w, jnp.float32)

    if A0_edge_index.dtype in (jnp.int64, jnp.uint64):
        a0 = lax.bitcast_convert_type(A0_edge_index, jnp.int32).reshape(-1)
        an = lax.bitcast_convert_type(An_edge_index, jnp.int32).reshape(-1)
        interleaved = True
    else:
        a0 = A0_edge_index.astype(jnp.int32).reshape(-1)
        an = An_edge_index.astype(jnp.int32).reshape(-1)
        interleaved = False

    out = _make_sc_fusion(n_out, interleaved, 10000)(a0, an, wv)
    return out.reshape(shape)


# speedup vs baseline: 1.0526x; 1.0526x over previous
import functools

from jax._src import config as _jax_config

import jax
import jax.numpy as jnp
from jax import lax
from jax.experimental import pallas as pl
from jax.experimental.pallas import tpu as pltpu

_ROWS, _COLS = 25000, 1024
_BM = 200


def _tc_body(a0_ref, an_ref, w_ref, o_ref):
    w = w_ref[0]
    row = jax.lax.broadcasted_iota(jnp.int32, (_COLS, _COLS // 2), 0)
    col = jax.lax.broadcasted_iota(jnp.int32, (_COLS, _COLS // 2), 1)
    sel = (row == 2 * col).astype(jnp.float32)
    a0 = jnp.dot(a0_ref[...].astype(jnp.float32), sel,
                 preferred_element_type=jnp.float32)
    an = jnp.dot(an_ref[...].astype(jnp.float32), sel,
                 preferred_element_type=jnp.float32)
    o_ref[...] = a0 * w + an * (1.0 - w)


def _make_tc_fusion(n_out: int):
    assert n_out * 2 == _ROWS * _COLS
    grid = (_ROWS // _BM,)
    return pl.pallas_call(
        _tc_body,
        grid=grid,
        in_specs=[
            pl.BlockSpec((_BM, _COLS), lambda i: (i, 0)),
            pl.BlockSpec((_BM, _COLS), lambda i: (i, 0)),
            pl.BlockSpec(memory_space=pltpu.SMEM),
        ],
        out_specs=pl.BlockSpec((_BM, _COLS // 2), lambda i: (i, 0)),
        out_shape=jax.ShapeDtypeStruct((_ROWS, _COLS // 2), jnp.float32),
    )


def kernel(A0_edge_index, An_edge_index, ver, p):
    with _jax_config.enable_x64(False):
        return _kernel_impl(A0_edge_index, An_edge_index, ver, p)


def _kernel_impl(A0_edge_index, An_edge_index, ver, p):
    del ver
    shape = A0_edge_index.shape
    n_out = shape[0] * shape[1]
    w = lax.stop_gradient(jax.nn.sigmoid(p)).astype(jnp.float32)
    wv = jnp.full((1,), w, jnp.float32)

    a0 = lax.bitcast_convert_type(A0_edge_index, jnp.int32).reshape(_ROWS, _COLS)
    an = lax.bitcast_convert_type(An_edge_index, jnp.int32).reshape(_ROWS, _COLS)

    out = _make_tc_fusion(n_out)(a0, an, wv)
    return out.reshape(shape)


# speedup vs baseline: 15.2280x; 14.4668x over previous
import functools

from jax._src import config as _jax_config

import jax
import jax.numpy as jnp
from jax import lax
from jax.experimental import pallas as pl
from jax.experimental.pallas import tpu as pltpu

_ROWS, _COLS = 25000, 512
_BM = 1000


def _tc_body(a0_ref, an_ref, w_ref, o_ref):
    w = w_ref[0]
    a0 = a0_ref[...].astype(jnp.float32)
    an = an_ref[...].astype(jnp.float32)
    o_ref[...] = a0 * w + an * (1.0 - w)


def _make_tc_fusion():
    grid = (_ROWS // _BM,)
    return pl.pallas_call(
        _tc_body,
        grid=grid,
        in_specs=[
            pl.BlockSpec((_BM, _COLS), lambda i: (i, 0)),
            pl.BlockSpec((_BM, _COLS), lambda i: (i, 0)),
            pl.BlockSpec(memory_space=pltpu.SMEM),
        ],
        out_specs=pl.BlockSpec((_BM, _COLS), lambda i: (i, 0)),
        out_shape=jax.ShapeDtypeStruct((_ROWS, _COLS), jnp.float32),
        compiler_params=pltpu.CompilerParams(
            allow_input_fusion=[True, True, False],
        ),
    )


def kernel(A0_edge_index, An_edge_index, ver, p):
    with _jax_config.enable_x64(False):
        return _kernel_impl(A0_edge_index, An_edge_index, ver, p)


def _kernel_impl(A0_edge_index, An_edge_index, ver, p):
    del ver
    shape = A0_edge_index.shape
    w = lax.stop_gradient(jax.nn.sigmoid(p)).astype(jnp.float32)
    wv = jnp.full((1,), w, jnp.float32)

    a0 = A0_edge_index.astype(jnp.int32).reshape(_ROWS, _COLS)
    an = An_edge_index.astype(jnp.int32).reshape(_ROWS, _COLS)

    out = _make_tc_fusion()(a0, an, wv)
    return out.reshape(shape)


# speedup vs baseline: 21.6483x; 1.4216x over previous
import functools

from jax._src import config as _jax_config

import jax
import jax.numpy as jnp
from jax import lax
from jax.experimental import pallas as pl
from jax.experimental.pallas import tpu as pltpu

_BN = 128000


def _tc_body(a0_ref, an_ref, w_ref, o_ref):
    w = w_ref[0]
    a0 = a0_ref[...].astype(jnp.float32)
    an = an_ref[...].astype(jnp.float32)
    o_ref[...] = a0 * w + an * (1.0 - w)


def _make_tc_fusion(rows: int, cols: int):
    assert cols % _BN == 0
    grid = (cols // _BN,)
    return pl.pallas_call(
        _tc_body,
        grid=grid,
        in_specs=[
            pl.BlockSpec((rows, _BN), lambda i: (0, i)),
            pl.BlockSpec((rows, _BN), lambda i: (0, i)),
            pl.BlockSpec(memory_space=pltpu.SMEM),
        ],
        out_specs=pl.BlockSpec((rows, _BN), lambda i: (0, i)),
        out_shape=jax.ShapeDtypeStruct((rows, cols), jnp.float32),
        compiler_params=pltpu.CompilerParams(
            allow_input_fusion=[True, True, False],
        ),
    )


def kernel(A0_edge_index, An_edge_index, ver, p):
    with _jax_config.enable_x64(False):
        return _kernel_impl(A0_edge_index, An_edge_index, ver, p)


def _kernel_impl(A0_edge_index, An_edge_index, ver, p):
    del ver
    rows, cols = A0_edge_index.shape
    w = lax.stop_gradient(jax.nn.sigmoid(p)).astype(jnp.float32)
    wv = jnp.full((1,), w, jnp.float32)

    a0 = A0_edge_index.astype(jnp.uint32)
    an = An_edge_index.astype(jnp.uint32)

    return _make_tc_fusion(rows, cols)(a0, an, wv)


# speedup vs baseline: 22.0094x; 1.0167x over previous
import functools

from jax._src import config as _jax_config

import jax
import jax.numpy as jnp
from jax import lax
from jax.experimental import pallas as pl
from jax.experimental.pallas import tpu as pltpu

_BN = 256000


def _tc_body(a0_ref, an_ref, w_ref, o_ref):
    w = w_ref[0]
    a0 = a0_ref[...].astype(jnp.float32)
    an = an_ref[...].astype(jnp.float32)
    o_ref[...] = a0 * w + an * (1.0 - w)


def _make_tc_fusion(rows: int, cols: int):
    assert cols % _BN == 0
    grid = (cols // _BN,)
    return pl.pallas_call(
        _tc_body,
        grid=grid,
        in_specs=[
            pl.BlockSpec((rows, _BN), lambda i: (0, i)),
            pl.BlockSpec((rows, _BN), lambda i: (0, i)),
            pl.BlockSpec(memory_space=pltpu.SMEM),
        ],
        out_specs=pl.BlockSpec((rows, _BN), lambda i: (0, i)),
        out_shape=jax.ShapeDtypeStruct((rows, cols), jnp.float32),
        compiler_params=pltpu.CompilerParams(
            allow_input_fusion=[True, True, False],
        ),
    )


def kernel(A0_edge_index, An_edge_index, ver, p):
    with _jax_config.enable_x64(False):
        return _kernel_impl(A0_edge_index, An_edge_index, ver, p)


def _kernel_impl(A0_edge_index, An_edge_index, ver, p):
    del ver
    rows, cols = A0_edge_index.shape
    w = lax.stop_gradient(jax.nn.sigmoid(p)).astype(jnp.float32)
    wv = jnp.full((1,), w, jnp.float32)

    a0 = A0_edge_index.astype(jnp.uint32)
    an = An_edge_index.astype(jnp.uint32)

    return _make_tc_fusion(rows, cols)(a0, an, wv)


# speedup vs baseline: 22.0535x; 1.0020x over previous
import functools

from jax._src import config as _jax_config

import jax
import jax.numpy as jnp
from jax import lax
from jax.experimental import pallas as pl
from jax.experimental.pallas import tpu as pltpu

_BN = 640000


def _tc_body(a0_ref, an_ref, w_ref, o_ref):
    w = w_ref[0]
    a0 = a0_ref[...].astype(jnp.float32)
    an = an_ref[...].astype(jnp.float32)
    o_ref[...] = a0 * w + an * (1.0 - w)


def _make_tc_fusion(rows: int, cols: int):
    assert cols % _BN == 0
    grid = (cols // _BN,)
    return pl.pallas_call(
        _tc_body,
        grid=grid,
        in_specs=[
            pl.BlockSpec((rows, _BN), lambda i: (0, i)),
            pl.BlockSpec((rows, _BN), lambda i: (0, i)),
            pl.BlockSpec(memory_space=pltpu.SMEM),
        ],
        out_specs=pl.BlockSpec((rows, _BN), lambda i: (0, i)),
        out_shape=jax.ShapeDtypeStruct((rows, cols), jnp.float32),
        compiler_params=pltpu.CompilerParams(
            allow_input_fusion=[True, True, False],
        ),
    )


def kernel(A0_edge_index, An_edge_index, ver, p):
    with _jax_config.enable_x64(False):
        return _kernel_impl(A0_edge_index, An_edge_index, ver, p)


def _kernel_impl(A0_edge_index, An_edge_index, ver, p):
    del ver
    rows, cols = A0_edge_index.shape
    w = lax.stop_gradient(jax.nn.sigmoid(p)).astype(jnp.float32)
    wv = jnp.full((1,), w, jnp.float32)

    a0 = A0_edge_index.astype(jnp.uint32)
    an = An_edge_index.astype(jnp.uint32)

    return _make_tc_fusion(rows, cols)(a0, an, wv)


# speedup vs baseline: 22.0748x; 1.0010x over previous
import functools

from jax._src import config as _jax_config

import jax
import jax.numpy as jnp
from jax import lax
from jax.experimental import pallas as pl
from jax.experimental.pallas import tpu as pltpu

_BN = 640000


def _tc_body(a0_ref, an_ref, w_ref, o_ref):
    w = w_ref[0]
    a0 = a0_ref[...].astype(jnp.float32)
    an = an_ref[...].astype(jnp.float32)
    o_ref[...] = a0 * w + an * (1.0 - w)


def _make_tc_fusion(rows: int, cols: int, bn: int):
    grid = (cols // bn,)
    return pl.pallas_call(
        _tc_body,
        grid=grid,
        in_specs=[
            pl.BlockSpec((rows, bn), lambda i: (0, i)),
            pl.BlockSpec((rows, bn), lambda i: (0, i)),
            pl.BlockSpec(memory_space=pltpu.SMEM),
        ],
        out_specs=pl.BlockSpec((rows, bn), lambda i: (0, i)),
        out_shape=jax.ShapeDtypeStruct((rows, cols), jnp.float32),
    )


def kernel(A0_edge_index, An_edge_index, ver, p):
    with _jax_config.enable_x64(False):
        return _kernel_impl(A0_edge_index, An_edge_index, ver, p)


def _kernel_impl(A0_edge_index, An_edge_index, ver, p):
    del ver
    rows, cols = A0_edge_index.shape
    bn = _BN if cols % _BN == 0 else cols
    w = lax.stop_gradient(jax.nn.sigmoid(p)).astype(jnp.float32)
    wv = jnp.full((1,), w, jnp.float32)

    a0 = A0_edge_index.astype(jnp.uint32)
    an = An_edge_index.astype(jnp.uint32)

    return _make_tc_fusion(rows, cols, bn)(a0, an, wv)
